# baseline (speedup 1.0000x reference)
import functools

import jax
import jax.numpy as jnp
from jax import lax
from jax.experimental import pallas as pl
from jax.experimental.pallas import tpu as pltpu
from jax.experimental.pallas import tpu_sc as plsc

BATCH = 4
SEQ = 2048
HIDDEN = 1024
D_EMB = HIDDEN // 2
D_TYPE = HIDDEN // 4
COORD_VOCAB = 1004
TYPE_VOCAB = 16
EPS = 1e-12

N_TOK = BATCH * SEQ
N_SC = N_TOK // 2
NC = 2
NS = 16
NW = NC * NS
TPW = N_SC // NW
C = 8
NCHUNK = TPW // C
ROWS = 16 * C

_A = (0, 0, 1, 1, 4, 4, 5, 5)
_B = (2, 3, 2, 3, 6, 7, 6, 7)


def _sc_body(table_hbm, idx_hbm, dv_hbm, out_hbm, idx_v, dv_v,
             rows_a, rows_b, out_a, out_b, sem_a, sem_b, osem_a, osem_b):
    cid = lax.axis_index("c")
    sid = lax.axis_index("s")
    wid = sid * NC + cid
    tok0 = wid * TPW
    pltpu.sync_copy(idx_hbm.at[pl.ds(tok0 * 16, TPW * 16)], idx_v)
    pltpu.sync_copy(dv_hbm.at[pl.ds(tok0 * 8, TPW * 8)], dv_v)

    def start_gather(g, rows, sem):
        pltpu.async_copy(table_hbm.at[idx_v.at[pl.ds(g * ROWS, ROWS)]], rows, sem)

    def wait_gather(g, rows, sem):
        pltpu.make_async_copy(
            table_hbm.at[idx_v.at[pl.ds(g * ROWS, ROWS)]], rows, sem
        ).wait()

    def out_dma(g, out_v, osem):
        return pltpu.make_async_copy(
            out_v, out_hbm.at[pl.ds(tok0 + g * C, C)], osem
        )

    def compute_chunk(g, rows, out_v):
        def tok_body(t, carry2):
            dv_base = g * (C * 8) + t * 8
            wb = []
            for k in range(8):
                ia = jnp.full((16,), dv_base + _A[k], dtype=jnp.int32)
                ib = jnp.full((16,), dv_base + _B[k], dtype=jnp.int32)
                wb.append(plsc.load_gather(dv_v, [ia]) * plsc.load_gather(dv_v, [ib]))
            r0 = t * 16
            for grp in range(D_EMB // 32):
                sl = pl.ds(grp * 16, 16)
                acc_a = None
                acc_b = None
                for k in range(8):
                    x32 = plsc.bitcast(rows[r0 + k, sl], jnp.bfloat16)
                    y32 = plsc.bitcast(rows[r0 + 8 + k, sl], jnp.bfloat16)
                    sa, sb = plsc.unpack(x32 + y32, format=plsc.PackFormat.INTERLEAVED)
                    ta = wb[k] * sa
                    tb = wb[k] * sb
                    acc_a = ta if acc_a is None else acc_a + ta
                    acc_b = tb if acc_b is None else acc_b + tb
                out_v[t, pl.ds(grp * 32, 16)] = acc_a
                out_v[t, pl.ds(grp * 32 + 16, 16)] = acc_b
            return carry2

        lax.fori_loop(0, C, tok_body, 0)

    start_gather(0, rows_a, sem_a)
    start_gather(1, rows_b, sem_b)

    def pipe_body(gp, carry):
        for b in range(2):
            rows = rows_a if b == 0 else rows_b
            sem = sem_a if b == 0 else sem_b
            out_v = out_a if b == 0 else out_b
            osem = osem_a if b == 0 else osem_b
            g = gp * 2 + b
            wait_gather(g, rows, sem)

            @pl.when(gp >= 1)
            def _():
                out_dma(g, out_v, osem).wait()

            compute_chunk(g, rows, out_v)
            out_dma(g, out_v, osem).start()

            @pl.when(g + 2 < NCHUNK)
            def _():
                start_gather(g + 2, rows, sem)

        return carry

    lax.fori_loop(0, NCHUNK // 2, pipe_body, 0)
    out_dma(NCHUNK - 2, out_a, osem_a).wait()
    out_dma(NCHUNK - 1, out_b, osem_b).wait()


_sc_gather = functools.partial(
    pl.kernel,
    out_type=jax.ShapeDtypeStruct((N_SC, D_EMB), jnp.float32),
    mesh=plsc.VectorSubcoreMesh(core_axis_name="c", subcore_axis_name="s"),
    scratch_types=[
        pltpu.VMEM((TPW * 16,), jnp.int32),
        pltpu.VMEM((TPW * 8,), jnp.float32),
        pltpu.VMEM((ROWS, D_EMB // 2), jnp.int32),
        pltpu.VMEM((ROWS, D_EMB // 2), jnp.int32),
        pltpu.VMEM((C, D_EMB), jnp.float32),
        pltpu.VMEM((C, D_EMB), jnp.float32),
        pltpu.SemaphoreType.DMA,
        pltpu.SemaphoreType.DMA,
        pltpu.SemaphoreType.DMA,
        pltpu.SemaphoreType.DMA,
    ],
    compiler_params=pltpu.CompilerParams(needs_layout_passes=False),
)(_sc_body)


_T = 2048


def _tc_core(emb_ref, idr_ref, idc_ref, trow_ref, tcol_ref, pos_ref, g_ref, b_ref, o_ref):
    f32 = jnp.float32
    idr = idr_ref[0, 0, :]
    idc = idc_ref[0, 0, :]
    iota = lax.broadcasted_iota(jnp.int32, (_T, TYPE_VOCAB), 1)
    ohr = (idr[:, None] == iota).astype(f32)
    ohc = (idc[:, None] == iota).astype(f32)
    ttr = jnp.dot(ohr, trow_ref[...], preferred_element_type=f32)
    ttc = jnp.dot(ohc, tcol_ref[...], preferred_element_type=f32)
    p1 = ttr + pos_ref[:, 0:D_TYPE]
    p2 = ttc + pos_ref[:, D_TYPE:2 * D_TYPE]
    p3 = emb_ref[...] + pos_ref[:, 2 * D_TYPE:]
    s = jnp.sum(p1, axis=-1) + jnp.sum(p2, axis=-1) + jnp.sum(p3, axis=-1)
    sq = jnp.sum(p1 * p1, axis=-1) + jnp.sum(p2 * p2, axis=-1) + jnp.sum(p3 * p3, axis=-1)
    mean = s * (1.0 / HIDDEN)
    var = sq * (1.0 / HIDDEN) - mean * mean
    inv = lax.rsqrt(var + EPS)
    mean_ = mean[:, None]
    inv_ = inv[:, None]
    o_ref[:, 0:D_TYPE] = (p1 - mean_) * inv_ * g_ref[0, 0:D_TYPE][None, :] + b_ref[0, 0:D_TYPE][None, :]
    o_ref[:, D_TYPE:2 * D_TYPE] = (p2 - mean_) * inv_ * g_ref[0, D_TYPE:2 * D_TYPE][None, :] + b_ref[0, D_TYPE:2 * D_TYPE][None, :]
    o_ref[:, 2 * D_TYPE:] = (p3 - mean_) * inv_ * g_ref[0, 2 * D_TYPE:][None, :] + b_ref[0, 2 * D_TYPE:][None, :]


def _tc_body8(emb_ref, idr_ref, idc_ref, trow_ref, tcol_ref, pos_ref, g_ref, b_ref, o_ref):
    _tc_core(emb_ref, idr_ref, idc_ref, trow_ref, tcol_ref, pos_ref, g_ref, b_ref, o_ref)


def _tc_body9(emb_ref, idr_ref, idc_ref, trow_ref, tcol_ref, pos_ref, g_ref, b_ref, prev_ref, o_ref):
    _tc_core(emb_ref, idr_ref, idc_ref, trow_ref, tcol_ref, pos_ref, g_ref, b_ref, o_ref)


def _tc_call_half(h, emb_h, idr3_h, idc3_h, trow, tcol, pos, g2, b2, prev):
    nblk = N_SC // _T
    in_specs = [
        pl.BlockSpec((_T, D_EMB), lambda i: (i, 0)),
        pl.BlockSpec((1, 1, _T), lambda i: (i, 0, 0)),
        pl.BlockSpec((1, 1, _T), lambda i: (i, 0, 0)),
        pl.BlockSpec((TYPE_VOCAB, D_TYPE), lambda i: (0, 0)),
        pl.BlockSpec((TYPE_VOCAB, D_TYPE), lambda i: (0, 0)),
        pl.BlockSpec((_T, HIDDEN), lambda i: (0, 0)),
        pl.BlockSpec((1, HIDDEN), lambda i: (0, 0)),
        pl.BlockSpec((1, HIDDEN), lambda i: (0, 0)),
    ]
    args = [emb_h, idr3_h, idc3_h, trow, tcol, pos, g2, b2]
    kwargs = {}
    body = _tc_body8
    if prev is not None:
        in_specs.append(pl.BlockSpec((8, 128), lambda i: (0, 0)))
        args.append(prev)
        kwargs["input_output_aliases"] = {8: 0}
        body = _tc_body9
    return pl.pallas_call(
        body,
        grid=(nblk,),
        in_specs=in_specs,
        out_specs=pl.BlockSpec((_T, HIDDEN), lambda i, h=h: (i + h * (N_SC // _T), 0)),
        out_shape=jax.ShapeDtypeStruct((N_TOK, HIDDEN), jnp.float32),
        **kwargs,
    )(*args)


def kernel(input_coord_ids, input_delta_vars, token_type_ids_row, token_type_ids_col,
           position_ids, coord_x_table, coord_y_table, type_row_table, type_col_table,
           pos_table, ln_gamma, ln_beta):
    ids = input_coord_ids.astype(jnp.int32).reshape(N_TOK, 8, 2)
    flat_idx = jnp.concatenate(
        [ids[:, :, 0], ids[:, :, 1] + COORD_VOCAB], axis=1
    ).reshape(N_TOK * 16)
    dv = input_delta_vars.astype(jnp.float32).reshape(N_TOK * 8)
    table = jnp.concatenate([coord_x_table, coord_y_table], axis=0)
    table_bf = (
        table.astype(jnp.bfloat16)
        .reshape(2 * COORD_VOCAB, D_EMB // 32, 2, 16)
        .transpose(0, 1, 3, 2)
        .reshape(2 * COORD_VOCAB, D_EMB // 2, 2)
    )
    table_i32 = lax.bitcast_convert_type(table_bf, jnp.int32)

    emb0 = _sc_gather(table_i32, flat_idx[: N_SC * 16], dv[: N_SC * 8])
    emb1 = _sc_gather(table_i32, flat_idx[N_SC * 16:], dv[N_SC * 8:])

    idr3 = token_type_ids_row.astype(jnp.int32).reshape(N_TOK // _T, 1, _T)
    idc3 = token_type_ids_col.astype(jnp.int32).reshape(N_TOK // _T, 1, _T)
    g2 = ln_gamma.reshape(1, HIDDEN)
    b2 = ln_beta.reshape(1, HIDDEN)
    hb = N_SC // _T
    out0 = _tc_call_half(0, emb0, idr3[:hb], idc3[:hb], type_row_table,
                         type_col_table, pos_table, g2, b2, None)
    out = _tc_call_half(1, emb1, idr3[hb:], idc3[hb:], type_row_table,
                        type_col_table, pos_table, g2, b2, out0)
    return out.reshape(BATCH, SEQ, HIDDEN)


# speedup vs baseline: 1.1413x; 1.1413x over previous
import functools

import jax
import jax.numpy as jnp
from jax import lax
from jax.experimental import pallas as pl
from jax.experimental.pallas import tpu as pltpu
from jax.experimental.pallas import tpu_sc as plsc

BATCH = 4
SEQ = 2048
HIDDEN = 1024
D_EMB = HIDDEN // 2
D_TYPE = HIDDEN // 4
COORD_VOCAB = 1004
TYPE_VOCAB = 16
EPS = 1e-12

N_TOK = BATCH * SEQ
NC = 2
NS = 16
NW = NC * NS
TPW = N_TOK // NW
C = 8
NCHUNK = TPW // C
ROWS = 16 * C

_A = (0, 0, 1, 1, 4, 4, 5, 5)
_B = (2, 3, 2, 3, 6, 7, 6, 7)


def _sc_body(table_hbm, idx_hbm, dv_hbm, out_hbm, idx_v, dv_v,
             rows_a, rows_b, out_a, out_b, sem_a, sem_b, osem_a, osem_b):
    cid = lax.axis_index("c")
    sid = lax.axis_index("s")
    wid = sid * NC + cid
    tok0 = wid * TPW
    pltpu.sync_copy(idx_hbm.at[pl.ds(tok0 * 16, TPW * 16)], idx_v)
    pltpu.sync_copy(dv_hbm.at[pl.ds(tok0 * 8, TPW * 8)], dv_v)

    def start_gather(g, rows, sem):
        pltpu.async_copy(table_hbm.at[idx_v.at[pl.ds(g * ROWS, ROWS)]], rows, sem)

    def wait_gather(g, rows, sem):
        pltpu.make_async_copy(
            table_hbm.at[idx_v.at[pl.ds(g * ROWS, ROWS)]], rows, sem
        ).wait()

    def out_dma(g, out_v, osem):
        return pltpu.make_async_copy(
            out_v, out_hbm.at[pl.ds(tok0 + g * C, C)], osem
        )

    def compute_chunk(g, rows, out_v):
        def tok_body(t, carry2):
            dv_base = g * (C * 8) + t * 8
            wb = []
            for k in range(8):
                ia = jnp.full((16,), dv_base + _A[k], dtype=jnp.int32)
                ib = jnp.full((16,), dv_base + _B[k], dtype=jnp.int32)
                w = plsc.load_gather(dv_v, [ia]) * plsc.load_gather(dv_v, [ib])
                wb.append(plsc.pack(w, w, format=plsc.PackFormat.INTERLEAVED))
            r0 = t * 16
            for grp in range(D_EMB // 32):
                sl = pl.ds(grp * 16, 16)
                acc_a = None
                acc_b = None
                for k in range(8):
                    x32 = plsc.bitcast(rows[r0 + k, sl], jnp.bfloat16)
                    y32 = plsc.bitcast(rows[r0 + 8 + k, sl], jnp.bfloat16)
                    ta, tb = plsc.unpack((x32 + y32) * wb[k],
                                         format=plsc.PackFormat.INTERLEAVED)
                    acc_a = ta if acc_a is None else acc_a + ta
                    acc_b = tb if acc_b is None else acc_b + tb
                out_v[t, pl.ds(grp * 32, 16)] = acc_a
                out_v[t, pl.ds(grp * 32 + 16, 16)] = acc_b
            return carry2

        lax.fori_loop(0, C, tok_body, 0)

    start_gather(0, rows_a, sem_a)
    start_gather(1, rows_b, sem_b)

    def pipe_body(gp, carry):
        for b in range(2):
            rows = rows_a if b == 0 else rows_b
            sem = sem_a if b == 0 else sem_b
            out_v = out_a if b == 0 else out_b
            osem = osem_a if b == 0 else osem_b
            g = gp * 2 + b
            wait_gather(g, rows, sem)

            @pl.when(gp >= 1)
            def _():
                out_dma(g, out_v, osem).wait()

            compute_chunk(g, rows, out_v)
            out_dma(g, out_v, osem).start()

            @pl.when(g + 2 < NCHUNK)
            def _():
                start_gather(g + 2, rows, sem)

        return carry

    lax.fori_loop(0, NCHUNK // 2, pipe_body, 0)
    out_dma(NCHUNK - 2, out_a, osem_a).wait()
    out_dma(NCHUNK - 1, out_b, osem_b).wait()


_sc_gather = functools.partial(
    pl.kernel,
    out_type=jax.ShapeDtypeStruct((N_TOK, D_EMB), jnp.float32),
    mesh=plsc.VectorSubcoreMesh(core_axis_name="c", subcore_axis_name="s"),
    scratch_types=[
        pltpu.VMEM((TPW * 16,), jnp.int32),
        pltpu.VMEM((TPW * 8,), jnp.float32),
        pltpu.VMEM((ROWS, D_EMB // 2), jnp.int32),
        pltpu.VMEM((ROWS, D_EMB // 2), jnp.int32),
        pltpu.VMEM((C, D_EMB), jnp.float32),
        pltpu.VMEM((C, D_EMB), jnp.float32),
        pltpu.SemaphoreType.DMA,
        pltpu.SemaphoreType.DMA,
        pltpu.SemaphoreType.DMA,
        pltpu.SemaphoreType.DMA,
    ],
    compiler_params=pltpu.CompilerParams(needs_layout_passes=False),
)(_sc_body)


_T = 2048


def _tc_body(emb_ref, idr_ref, idc_ref, trow_ref, tcol_ref, pos_ref, g_ref, b_ref, o_ref):
    f32 = jnp.float32
    idr = idr_ref[0, 0, :]
    idc = idc_ref[0, 0, :]
    iota = lax.broadcasted_iota(jnp.int32, (_T, TYPE_VOCAB), 1)
    ohr = (idr[:, None] == iota).astype(f32)
    ohc = (idc[:, None] == iota).astype(f32)
    ttr = jnp.dot(ohr, trow_ref[...], preferred_element_type=f32)
    ttc = jnp.dot(ohc, tcol_ref[...], preferred_element_type=f32)
    p1 = ttr + pos_ref[:, 0:D_TYPE]
    p2 = ttc + pos_ref[:, D_TYPE:2 * D_TYPE]
    p3 = emb_ref[...] + pos_ref[:, 2 * D_TYPE:]
    s = jnp.sum(p1, axis=-1) + jnp.sum(p2, axis=-1) + jnp.sum(p3, axis=-1)
    sq = jnp.sum(p1 * p1, axis=-1) + jnp.sum(p2 * p2, axis=-1) + jnp.sum(p3 * p3, axis=-1)
    mean = s * (1.0 / HIDDEN)
    var = sq * (1.0 / HIDDEN) - mean * mean
    inv = lax.rsqrt(var + EPS)
    mean_ = mean[:, None]
    inv_ = inv[:, None]
    o_ref[:, 0:D_TYPE] = (p1 - mean_) * inv_ * g_ref[0, 0:D_TYPE][None, :] + b_ref[0, 0:D_TYPE][None, :]
    o_ref[:, D_TYPE:2 * D_TYPE] = (p2 - mean_) * inv_ * g_ref[0, D_TYPE:2 * D_TYPE][None, :] + b_ref[0, D_TYPE:2 * D_TYPE][None, :]
    o_ref[:, 2 * D_TYPE:] = (p3 - mean_) * inv_ * g_ref[0, 2 * D_TYPE:][None, :] + b_ref[0, 2 * D_TYPE:][None, :]


def _tc_call(emb, idr3, idc3, trow, tcol, pos, g2, b2):
    nblk = N_TOK // _T
    sblk = SEQ // _T
    return pl.pallas_call(
        _tc_body,
        grid=(nblk,),
        in_specs=[
            pl.BlockSpec((_T, D_EMB), lambda i: (i, 0)),
            pl.BlockSpec((1, 1, _T), lambda i: (i, 0, 0)),
            pl.BlockSpec((1, 1, _T), lambda i: (i, 0, 0)),
            pl.BlockSpec((TYPE_VOCAB, D_TYPE), lambda i: (0, 0)),
            pl.BlockSpec((TYPE_VOCAB, D_TYPE), lambda i: (0, 0)),
            pl.BlockSpec((_T, HIDDEN), lambda i: (0, 0)),
            pl.BlockSpec((1, HIDDEN), lambda i: (0, 0)),
            pl.BlockSpec((1, HIDDEN), lambda i: (0, 0)),
        ],
        out_specs=pl.BlockSpec((_T, HIDDEN), lambda i: (i, 0)),
        out_shape=jax.ShapeDtypeStruct((N_TOK, HIDDEN), jnp.float32),
    )(emb, idr3, idc3, trow, tcol, pos, g2, b2)


def kernel(input_coord_ids, input_delta_vars, token_type_ids_row, token_type_ids_col,
           position_ids, coord_x_table, coord_y_table, type_row_table, type_col_table,
           pos_table, ln_gamma, ln_beta):
    ids = input_coord_ids.astype(jnp.int32).reshape(N_TOK, 8, 2)
    flat_idx = jnp.concatenate(
        [ids[:, :, 0], ids[:, :, 1] + COORD_VOCAB], axis=1
    ).reshape(N_TOK * 16)
    dv = input_delta_vars.astype(jnp.float32).reshape(N_TOK * 8)
    table = jnp.concatenate([coord_x_table, coord_y_table], axis=0)
    table_bf = (
        table.astype(jnp.bfloat16)
        .reshape(2 * COORD_VOCAB, D_EMB // 32, 2, 16)
        .transpose(0, 1, 3, 2)
        .reshape(2 * COORD_VOCAB, D_EMB // 2, 2)
    )
    table_i32 = lax.bitcast_convert_type(table_bf, jnp.int32)

    emb = _sc_gather(table_i32, flat_idx, dv)

    idr3 = token_type_ids_row.astype(jnp.int32).reshape(N_TOK // _T, 1, _T)
    idc3 = token_type_ids_col.astype(jnp.int32).reshape(N_TOK // _T, 1, _T)
    g2 = ln_gamma.reshape(1, HIDDEN)
    b2 = ln_beta.reshape(1, HIDDEN)
    out = _tc_call(emb, idr3, idc3, type_row_table, type_col_table, pos_table, g2, b2)
    return out.reshape(BATCH, SEQ, HIDDEN)


# speedup vs baseline: 1.1469x; 1.0049x over previous
import functools

import jax
import jax.numpy as jnp
from jax import lax
from jax.experimental import pallas as pl
from jax.experimental.pallas import tpu as pltpu
from jax.experimental.pallas import tpu_sc as plsc

BATCH = 4
SEQ = 2048
HIDDEN = 1024
D_EMB = HIDDEN // 2
D_TYPE = HIDDEN // 4
COORD_VOCAB = 1004
TYPE_VOCAB = 16
EPS = 1e-12

N_TOK = BATCH * SEQ
NC = 2
NS = 16
NW = NC * NS
TPW = N_TOK // NW
C = 8
NCHUNK = TPW // C
ROWS = 16 * C

_A = (0, 0, 1, 1, 4, 4, 5, 5)
_B = (2, 3, 2, 3, 6, 7, 6, 7)


def _sc_body(table_hbm, idx_hbm, dv_hbm, out_hbm, idx_v, dv_v,
             rows_a, rows_b, out_a, out_b, sem_a, sem_b, osem_a, osem_b):
    cid = lax.axis_index("c")
    sid = lax.axis_index("s")
    wid = sid * NC + cid
    tok0 = wid * TPW
    pltpu.sync_copy(idx_hbm.at[pl.ds(tok0 * 16, TPW * 16)], idx_v)
    pltpu.sync_copy(dv_hbm.at[pl.ds(tok0 * 8, TPW * 8)], dv_v)

    def start_gather(g, rows, sem):
        pltpu.async_copy(table_hbm.at[idx_v.at[pl.ds(g * ROWS, ROWS)]], rows, sem)

    def wait_gather(g, rows, sem):
        pltpu.make_async_copy(
            table_hbm.at[idx_v.at[pl.ds(g * ROWS, ROWS)]], rows, sem
        ).wait()

    def out_dma(g, out_v, osem):
        return pltpu.make_async_copy(
            out_v, out_hbm.at[pl.ds(tok0 + g * C, C)], osem
        )

    def compute_chunk(g, rows, out_v):
        def tok_body(t, carry2):
            dv_base = g * (C * 8) + t * 8
            wb = []
            for k in range(8):
                ia = jnp.full((16,), dv_base + _A[k], dtype=jnp.int32)
                ib = jnp.full((16,), dv_base + _B[k], dtype=jnp.int32)
                w = plsc.load_gather(dv_v, [ia]) * plsc.load_gather(dv_v, [ib])
                wb.append(plsc.pack(w, w, format=plsc.PackFormat.INTERLEAVED))
            r0 = t * 16
            for grp in range(D_EMB // 32):
                sl = pl.ds(grp * 16, 16)
                acc_a = None
                acc_b = None
                for k in range(0, 8, 2):
                    x0 = plsc.bitcast(rows[r0 + k, sl], jnp.bfloat16)
                    y0 = plsc.bitcast(rows[r0 + 8 + k, sl], jnp.bfloat16)
                    x1 = plsc.bitcast(rows[r0 + k + 1, sl], jnp.bfloat16)
                    y1 = plsc.bitcast(rows[r0 + 9 + k, sl], jnp.bfloat16)
                    pair = (x0 + y0) * wb[k] + (x1 + y1) * wb[k + 1]
                    ta, tb = plsc.unpack(pair, format=plsc.PackFormat.INTERLEAVED)
                    acc_a = ta if acc_a is None else acc_a + ta
                    acc_b = tb if acc_b is None else acc_b + tb
                out_v[t, pl.ds(grp * 32, 16)] = acc_a
                out_v[t, pl.ds(grp * 32 + 16, 16)] = acc_b
            return carry2

        lax.fori_loop(0, C, tok_body, 0)

    start_gather(0, rows_a, sem_a)
    start_gather(1, rows_b, sem_b)

    def pipe_body(gp, carry):
        for b in range(2):
            rows = rows_a if b == 0 else rows_b
            sem = sem_a if b == 0 else sem_b
            out_v = out_a if b == 0 else out_b
            osem = osem_a if b == 0 else osem_b
            g = gp * 2 + b
            wait_gather(g, rows, sem)

            @pl.when(gp >= 1)
            def _():
                out_dma(g, out_v, osem).wait()

            compute_chunk(g, rows, out_v)
            out_dma(g, out_v, osem).start()

            @pl.when(g + 2 < NCHUNK)
            def _():
                start_gather(g + 2, rows, sem)

        return carry

    lax.fori_loop(0, NCHUNK // 2, pipe_body, 0)
    out_dma(NCHUNK - 2, out_a, osem_a).wait()
    out_dma(NCHUNK - 1, out_b, osem_b).wait()


_sc_gather = functools.partial(
    pl.kernel,
    out_type=jax.ShapeDtypeStruct((N_TOK, D_EMB), jnp.float32),
    mesh=plsc.VectorSubcoreMesh(core_axis_name="c", subcore_axis_name="s"),
    scratch_types=[
        pltpu.VMEM((TPW * 16,), jnp.int32),
        pltpu.VMEM((TPW * 8,), jnp.float32),
        pltpu.VMEM((ROWS, D_EMB // 2), jnp.int32),
        pltpu.VMEM((ROWS, D_EMB // 2), jnp.int32),
        pltpu.VMEM((C, D_EMB), jnp.float32),
        pltpu.VMEM((C, D_EMB), jnp.float32),
        pltpu.SemaphoreType.DMA,
        pltpu.SemaphoreType.DMA,
        pltpu.SemaphoreType.DMA,
        pltpu.SemaphoreType.DMA,
    ],
    compiler_params=pltpu.CompilerParams(needs_layout_passes=False),
)(_sc_body)


_T = 2048


def _tc_body(emb_ref, idr_ref, idc_ref, trow_ref, tcol_ref, pos_ref, g_ref, b_ref, o_ref):
    f32 = jnp.float32
    idr = idr_ref[0, 0, :]
    idc = idc_ref[0, 0, :]
    iota = lax.broadcasted_iota(jnp.int32, (_T, TYPE_VOCAB), 1)
    ohr = (idr[:, None] == iota).astype(f32)
    ohc = (idc[:, None] == iota).astype(f32)
    ttr = jnp.dot(ohr, trow_ref[...], preferred_element_type=f32)
    ttc = jnp.dot(ohc, tcol_ref[...], preferred_element_type=f32)
    p1 = ttr + pos_ref[:, 0:D_TYPE]
    p2 = ttc + pos_ref[:, D_TYPE:2 * D_TYPE]
    p3 = emb_ref[...] + pos_ref[:, 2 * D_TYPE:]
    s = jnp.sum(p1, axis=-1) + jnp.sum(p2, axis=-1) + jnp.sum(p3, axis=-1)
    sq = jnp.sum(p1 * p1, axis=-1) + jnp.sum(p2 * p2, axis=-1) + jnp.sum(p3 * p3, axis=-1)
    mean = s * (1.0 / HIDDEN)
    var = sq * (1.0 / HIDDEN) - mean * mean
    inv = lax.rsqrt(var + EPS)
    mean_ = mean[:, None]
    inv_ = inv[:, None]
    o_ref[:, 0:D_TYPE] = (p1 - mean_) * inv_ * g_ref[0, 0:D_TYPE][None, :] + b_ref[0, 0:D_TYPE][None, :]
    o_ref[:, D_TYPE:2 * D_TYPE] = (p2 - mean_) * inv_ * g_ref[0, D_TYPE:2 * D_TYPE][None, :] + b_ref[0, D_TYPE:2 * D_TYPE][None, :]
    o_ref[:, 2 * D_TYPE:] = (p3 - mean_) * inv_ * g_ref[0, 2 * D_TYPE:][None, :] + b_ref[0, 2 * D_TYPE:][None, :]


def _tc_call(emb, idr3, idc3, trow, tcol, pos, g2, b2):
    nblk = N_TOK // _T
    sblk = SEQ // _T
    return pl.pallas_call(
        _tc_body,
        grid=(nblk,),
        in_specs=[
            pl.BlockSpec((_T, D_EMB), lambda i: (i, 0)),
            pl.BlockSpec((1, 1, _T), lambda i: (i, 0, 0)),
            pl.BlockSpec((1, 1, _T), lambda i: (i, 0, 0)),
            pl.BlockSpec((TYPE_VOCAB, D_TYPE), lambda i: (0, 0)),
            pl.BlockSpec((TYPE_VOCAB, D_TYPE), lambda i: (0, 0)),
            pl.BlockSpec((_T, HIDDEN), lambda i: (0, 0)),
            pl.BlockSpec((1, HIDDEN), lambda i: (0, 0)),
            pl.BlockSpec((1, HIDDEN), lambda i: (0, 0)),
        ],
        out_specs=pl.BlockSpec((_T, HIDDEN), lambda i: (i, 0)),
        out_shape=jax.ShapeDtypeStruct((N_TOK, HIDDEN), jnp.float32),
    )(emb, idr3, idc3, trow, tcol, pos, g2, b2)


def kernel(input_coord_ids, input_delta_vars, token_type_ids_row, token_type_ids_col,
           position_ids, coord_x_table, coord_y_table, type_row_table, type_col_table,
           pos_table, ln_gamma, ln_beta):
    ids = input_coord_ids.astype(jnp.int32).reshape(N_TOK, 8, 2)
    flat_idx = jnp.concatenate(
        [ids[:, :, 0], ids[:, :, 1] + COORD_VOCAB], axis=1
    ).reshape(N_TOK * 16)
    dv = input_delta_vars.astype(jnp.float32).reshape(N_TOK * 8)
    table = jnp.concatenate([coord_x_table, coord_y_table], axis=0)
    table_bf = (
        table.astype(jnp.bfloat16)
        .reshape(2 * COORD_VOCAB, D_EMB // 32, 2, 16)
        .transpose(0, 1, 3, 2)
        .reshape(2 * COORD_VOCAB, D_EMB // 2, 2)
    )
    table_i32 = lax.bitcast_convert_type(table_bf, jnp.int32)

    emb = _sc_gather(table_i32, flat_idx, dv)

    idr3 = token_type_ids_row.astype(jnp.int32).reshape(N_TOK // _T, 1, _T)
    idc3 = token_type_ids_col.astype(jnp.int32).reshape(N_TOK // _T, 1, _T)
    g2 = ln_gamma.reshape(1, HIDDEN)
    b2 = ln_beta.reshape(1, HIDDEN)
    out = _tc_call(emb, idr3, idc3, type_row_table, type_col_table, pos_table, g2, b2)
    return out.reshape(BATCH, SEQ, HIDDEN)
